# baseline (speedup 1.0000x reference)
import functools

import jax
import jax.numpy as jnp
from jax import lax
from jax.experimental import pallas as pl
from jax.experimental.pallas import tpu as pltpu
from jax.experimental.pallas import tpu_sc as plsc

NC = 2
NS = 16
L = 16
W = NC * NS

P = 2_000_000
HB = 128
NB = P // HB
BW = 488
CB = 61
NCHUNK = BW // CB
NXB = NB - W * BW
CPW = CB * 2 * HB
BH = 13
QH = BH * 2 * HB
QS = CPW - QH
V = 1_000_000
VSTG = 62496
NSP = VSTG // QS
SREM = VSTG - NSP * QS
MARGIN = 1.0
INV_P = 1.0 / P


@functools.partial(
    pl.kernel,
    out_type=jax.ShapeDtypeStruct((W, L), jnp.float32),
    mesh=plsc.VectorSubcoreMesh(
        core_axis_name="c", subcore_axis_name="s",
        num_cores=NC, num_subcores=NS),
    scratch_types=[
        pltpu.VMEM((CB, 2, HB), jnp.int32),
        pltpu.VMEM((QH,), jnp.int32),
        pltpu.VMEM((QS,), jnp.int32),
        pltpu.VMEM((QH,), jnp.float32),
        pltpu.VMEM((QH,), jnp.float32),
        pltpu.VMEM((QS,), jnp.float32),
        pltpu.VMEM((QS,), jnp.float32),
        pltpu.VMEM((L,), jnp.float32),
        pltpu.VMEM_SHARED((V,), jnp.float32),
        pltpu.SemaphoreType.DMA,
        [pltpu.SemaphoreType.DMA] * 2,
        [pltpu.SemaphoreType.DMA] * 2,
        pltpu.SemaphoreType.DMA,
    ],
)
def _sc_loss(idx3_hbm, outputs_hbm, out_hbm,
             idx3_v, idxh, idxs, valh0, valh1, vals0, vals1, acc_v, table_sh,
             isem, gsems, hsems, ssem):
    sid = lax.axis_index("s")
    wid = sid * NC + lax.axis_index("c")
    base = wid * BW
    valh = [valh0, valh1]
    vals = [vals0, vals1]

    def start_idx(c):
        pltpu.async_copy(idx3_hbm.at[pl.ds(base + c * CB, CB)], idx3_v, isem)

    def wait_idx():
        pltpu.make_async_copy(idx3_hbm.at[pl.ds(0, CB)], idx3_v, isem).wait()

    def deinterleave():
        def row_h(t, _):
            b = t * (2 * HB)
            for k in range(HB // L):
                idxh[pl.ds(b + k * L, L)] = idx3_v[t, 0, pl.ds(k * L, L)]
                idxh[pl.ds(b + HB + k * L, L)] = idx3_v[t, 1, pl.ds(k * L, L)]
            return 0
        lax.fori_loop(0, BH, row_h, 0)

        def row_s(t, _):
            b = (t - BH) * (2 * HB)
            for k in range(HB // L):
                idxs[pl.ds(b + k * L, L)] = idx3_v[t, 0, pl.ds(k * L, L)]
                idxs[pl.ds(b + HB + k * L, L)] = idx3_v[t, 1, pl.ds(k * L, L)]
            return 0
        lax.fori_loop(BH, CB, row_s, 0)

    def start_gather(b):
        pltpu.async_copy(outputs_hbm.at[idxh], valh[b], hsems[b])
        pltpu.async_copy(table_sh.at[idxs], vals[b], gsems[b])

    def wait_gather(b):
        pltpu.make_async_copy(outputs_hbm.at[idxh], valh[b], hsems[b]).wait()
        pltpu.make_async_copy(table_sh.at[idxs], vals[b], gsems[b]).wait()

    def accum(b, acc):
        def it_h(t, a):
            bb = t * (2 * HB)
            for k in range(HB // L):
                x0 = valh[b][pl.ds(bb + k * L, L)]
                x1 = valh[b][pl.ds(bb + HB + k * L, L)]
                a = a + jnp.maximum(MARGIN - x0 + x1, 0.0)
            return a
        acc = lax.fori_loop(0, BH, it_h, acc)

        def it_s(t, a):
            bb = t * (2 * HB)
            for k in range(HB // L):
                x0 = vals[b][pl.ds(bb + k * L, L)]
                x1 = vals[b][pl.ds(bb + HB + k * L, L)]
                a = a + jnp.maximum(MARGIN - x0 + x1, 0.0)
            return a
        return lax.fori_loop(0, CB - BH, it_s, acc)

    start_idx(0)

    for p in range(NSP):
        b = p % 2
        poff = pl.multiple_of(sid * VSTG + p * QS, 8)
        if p >= 2:
            pltpu.make_async_copy(vals[b], table_sh.at[pl.ds(0, QS)],
                                  ssem).wait()
        pltpu.async_copy(outputs_hbm.at[pl.ds(poff, QS)], vals[b],
                         gsems[b]).wait()
        pltpu.async_copy(vals[b], table_sh.at[pl.ds(poff, QS)], ssem)
    pltpu.make_async_copy(vals[0], table_sh.at[pl.ds(0, QS)], ssem).wait()
    pltpu.make_async_copy(vals[1], table_sh.at[pl.ds(0, QS)], ssem).wait()

    roff = pl.multiple_of(sid * VSTG + NSP * QS, 8)
    pltpu.async_copy(outputs_hbm.at[pl.ds(roff, SREM)],
                     vals0.at[pl.ds(0, SREM)], ssem).wait()
    pltpu.async_copy(vals0.at[pl.ds(0, SREM)],
                     table_sh.at[pl.ds(roff, SREM)], ssem).wait()

    @pl.when(sid == NS - 1)
    def _():
        poff = pl.multiple_of(NS * VSTG, 8)
        rem = V - NS * VSTG
        pltpu.async_copy(outputs_hbm.at[pl.ds(poff, rem)],
                         vals1.at[pl.ds(0, rem)], ssem).wait()
        pltpu.async_copy(vals1.at[pl.ds(0, rem)],
                         table_sh.at[pl.ds(poff, rem)], ssem).wait()

    plsc.subcore_barrier()

    acc = jnp.zeros((L,), jnp.float32)
    for c in range(NCHUNK):
        wait_idx()
        if c > 0:
            wait_gather((c - 1) % 2)
        deinterleave()
        if c < NCHUNK - 1:
            start_idx(c + 1)
        start_gather(c % 2)
        if c > 0:
            acc = accum((c - 1) % 2, acc)
    wait_gather((NCHUNK - 1) % 2)
    acc = accum((NCHUNK - 1) % 2, acc)

    @pl.when(wid < NXB)
    def _():
        pltpu.async_copy(idx3_hbm.at[pl.ds(W * BW + wid, 1)],
                         idx3_v.at[pl.ds(0, 1)], isem)
        pltpu.make_async_copy(idx3_hbm.at[pl.ds(0, 1)],
                              idx3_v.at[pl.ds(0, 1)], isem).wait()
        c0 = pltpu.async_copy(table_sh.at[idx3_v.at[0, 0]],
                              valh0.at[pl.ds(0, HB)], gsems[0])
        c1 = pltpu.async_copy(table_sh.at[idx3_v.at[0, 1]],
                              valh0.at[pl.ds(HB, HB)], gsems[0])
        c0.wait()
        c1.wait()
        a2 = acc
        for k in range(HB // L):
            x0 = valh0[pl.ds(k * L, L)]
            x1 = valh0[pl.ds(HB + k * L, L)]
            a2 = a2 + jnp.maximum(MARGIN - x0 + x1, 0.0)
        acc_v[...] = a2 * INV_P

    @pl.when(wid >= NXB)
    def _():
        acc_v[...] = acc * INV_P

    pltpu.sync_copy(acc_v, out_hbm.at[wid])


def kernel(outputs, mask):
    idx3 = (
        mask.astype(jnp.int32).T
        .reshape(2, NB, HB)
        .transpose(1, 0, 2)
    )
    parts = _sc_loss(idx3, outputs)
    return jnp.sum(parts)


# speedup vs baseline: 1.0557x; 1.0557x over previous
import functools

import jax
import jax.numpy as jnp
from jax import lax
from jax.experimental import pallas as pl
from jax.experimental.pallas import tpu as pltpu
from jax.experimental.pallas import tpu_sc as plsc

NC = 2
NS = 16
L = 16
W = NC * NS

P = 2_000_000
HB = 128
NB = P // HB
BW = 488
CB = 61
NCHUNK = BW // CB
NXB = NB - W * BW
CPW = CB * 2 * HB
V = 1_000_000
VSTG = 62496
NSP = VSTG // CPW
SREM = VSTG - NSP * CPW
MARGIN = 1.0
INV_P = 1.0 / P


@functools.partial(
    pl.kernel,
    out_type=jax.ShapeDtypeStruct((W, L), jnp.float32),
    mesh=plsc.VectorSubcoreMesh(
        core_axis_name="c", subcore_axis_name="s",
        num_cores=NC, num_subcores=NS),
    scratch_types=[
        pltpu.VMEM((CB, 2, HB), jnp.int32),
        pltpu.VMEM((CPW,), jnp.int32),
        pltpu.VMEM((CPW,), jnp.float32),
        pltpu.VMEM((CPW,), jnp.float32),
        pltpu.VMEM((L,), jnp.float32),
        pltpu.VMEM_SHARED((V,), jnp.float32),
        pltpu.SemaphoreType.DMA,
        [pltpu.SemaphoreType.DMA] * 2,
        pltpu.SemaphoreType.DMA,
    ],
)
def _sc_loss(idx3_hbm, outputs_hbm, out_hbm,
             idx3_v, idx1d, val_v0, val_v1, acc_v, table_sh,
             isem, gsems, ssem):
    sid = lax.axis_index("s")
    wid = sid * NC + lax.axis_index("c")
    base = wid * BW
    val = [val_v0, val_v1]

    def start_idx(c):
        pltpu.async_copy(idx3_hbm.at[pl.ds(base + c * CB, CB)], idx3_v, isem)

    def wait_idx():
        pltpu.make_async_copy(idx3_hbm.at[pl.ds(0, CB)], idx3_v, isem).wait()

    def deinterleave():
        def row(t, _):
            b = t * (2 * HB)
            for k in range(HB // L):
                idx1d[pl.ds(b + k * L, L)] = idx3_v[t, 0, pl.ds(k * L, L)]
                idx1d[pl.ds(b + HB + k * L, L)] = idx3_v[t, 1, pl.ds(k * L, L)]
            return 0
        lax.fori_loop(0, CB, row, 0)

    def start_gather(b):
        pltpu.async_copy(table_sh.at[idx1d], val[b], gsems[b])

    def wait_gather(b):
        pltpu.make_async_copy(table_sh.at[idx1d], val[b], gsems[b]).wait()

    def accum(b, acc):
        def it(t, a):
            bb = t * (2 * HB)
            for k in range(HB // L):
                x0 = val[b][pl.ds(bb + k * L, L)]
                x1 = val[b][pl.ds(bb + HB + k * L, L)]
                a = a + jnp.maximum(MARGIN - x0 + x1, 0.0)
            return a
        return lax.fori_loop(0, CB, it, acc)

    start_idx(0)

    for p in range(NSP):
        b = p % 2
        poff = pl.multiple_of(sid * VSTG + p * CPW, 8)
        if p >= 2:
            pltpu.make_async_copy(val[b], table_sh.at[pl.ds(0, CPW)],
                                  ssem).wait()
        pltpu.async_copy(outputs_hbm.at[pl.ds(poff, CPW)], val[b],
                         gsems[b]).wait()
        pltpu.async_copy(val[b], table_sh.at[pl.ds(poff, CPW)], ssem)
    pltpu.make_async_copy(val[0], table_sh.at[pl.ds(0, CPW)], ssem).wait()
    pltpu.make_async_copy(val[1], table_sh.at[pl.ds(0, CPW)], ssem).wait()

    roff = pl.multiple_of(sid * VSTG + NSP * CPW, 8)
    pltpu.async_copy(outputs_hbm.at[pl.ds(roff, SREM)],
                     val_v0.at[pl.ds(0, SREM)], ssem).wait()
    pltpu.async_copy(val_v0.at[pl.ds(0, SREM)],
                     table_sh.at[pl.ds(roff, SREM)], ssem).wait()

    @pl.when(sid == NS - 1)
    def _():
        poff = pl.multiple_of(NS * VSTG, 8)
        rem = V - NS * VSTG
        pltpu.async_copy(outputs_hbm.at[pl.ds(poff, rem)],
                         val_v1.at[pl.ds(0, rem)], ssem).wait()
        pltpu.async_copy(val_v1.at[pl.ds(0, rem)],
                         table_sh.at[pl.ds(poff, rem)], ssem).wait()

    plsc.subcore_barrier()

    acc = jnp.zeros((L,), jnp.float32)
    for c in range(NCHUNK):
        wait_idx()
        if c > 0:
            wait_gather((c - 1) % 2)
        deinterleave()
        if c < NCHUNK - 1:
            start_idx(c + 1)
        start_gather(c % 2)
        if c > 0:
            acc = accum((c - 1) % 2, acc)
    wait_gather((NCHUNK - 1) % 2)
    acc = accum((NCHUNK - 1) % 2, acc)

    @pl.when(wid < NXB)
    def _():
        pltpu.async_copy(idx3_hbm.at[pl.ds(W * BW + wid, 1)],
                         idx3_v.at[pl.ds(0, 1)], isem)
        pltpu.make_async_copy(idx3_hbm.at[pl.ds(0, 1)],
                              idx3_v.at[pl.ds(0, 1)], isem).wait()
        c0 = pltpu.async_copy(table_sh.at[idx3_v.at[0, 0]],
                              val_v0.at[pl.ds(0, HB)], gsems[0])
        c1 = pltpu.async_copy(table_sh.at[idx3_v.at[0, 1]],
                              val_v0.at[pl.ds(HB, HB)], gsems[0])
        c0.wait()
        c1.wait()
        a2 = acc
        for k in range(HB // L):
            x0 = val_v0[pl.ds(k * L, L)]
            x1 = val_v0[pl.ds(HB + k * L, L)]
            a2 = a2 + jnp.maximum(MARGIN - x0 + x1, 0.0)
        acc_v[...] = a2 * INV_P

    @pl.when(wid >= NXB)
    def _():
        acc_v[...] = acc * INV_P

    pltpu.sync_copy(acc_v, out_hbm.at[wid])


def kernel(outputs, mask):
    idx3 = (
        mask.astype(jnp.int32).T
        .reshape(2, NB, HB)
        .transpose(1, 0, 2)
    )
    parts = _sc_loss(idx3, outputs)
    return jnp.sum(parts)
